# Initial kernel scaffold; baseline (speedup 1.0000x reference)
import jax
import jax.numpy as jnp
from jax.experimental import pallas as pl


def kernel(points, features, enc0_W1, enc0_b1, enc0_W2, enc0_b2, enc1_W1, enc1_b1, enc1_W2, enc1_b2, enc2_W1, enc2_b1, enc2_W2, enc2_b2, enc3_W1, enc3_b1, enc3_W2, enc3_b2, up0_W1, up0_b1, up0_W2, up0_b2, up1_W1, up1_b1, up1_W2, up1_b2, up2_W1, up2_b1, up2_W2, up2_b2, up3_W1, up3_b1, up3_W2, up3_b2, lin0_W, lin0_b, lin1_W, lin1_b, lin2_W, lin2_b, mlp0_W, mlp0_b, mlp1_W, mlp1_b, mlp2_W, mlp2_b, mlp3_W, mlp3_b):
    raise NotImplementedError("write your pallas kernel here")


# speedup vs baseline: 7.5067x; 7.5067x over previous
import functools

import jax
import jax.numpy as jnp
from jax import lax
from jax.experimental import pallas as pl
from jax.experimental.pallas import tpu as pltpu
from jax.experimental.pallas import tpu_sc as plsc

F32 = jnp.float32
I32 = jnp.int32
K = 16
NW = 32


def _knn_body(qp_ref, sT_ref, idx_ref, d_ref, i4_ref, D_ref):
    q = qp_ref[...]
    sT = sT_ref[...]
    qsq = jnp.sum(q * q, axis=1, keepdims=True)
    ssq = jnp.sum(sT * sT, axis=0, keepdims=True)
    D = qsq + ssq - 2.0 * jnp.dot(q, sT, preferred_element_type=F32)
    D_ref[...] = D
    S = sT.shape[1]
    Qb = q.shape[0]
    cols = lax.broadcasted_iota(I32, (Qb, S), 1)
    idx_cols = []
    d_cols = []
    for _ in range(K):
        Dk = D_ref[...]
        mn = jnp.min(Dk, axis=1, keepdims=True)
        am = jnp.min(jnp.where(Dk <= mn, cols, S), axis=1, keepdims=True)
        idx_cols.append(am)
        d_cols.append(jnp.maximum(mn, 0.0))
        D_ref[...] = jnp.where(cols == am, jnp.inf, Dk)
    idx_ref[...] = jnp.concatenate(idx_cols, axis=1)
    d_ref[...] = jnp.concatenate(d_cols, axis=1)
    i4_ref[...] = jnp.concatenate(idx_cols[:4], axis=1)


def _knn(qp8, sT, Qb=256):
    Q = qp8.shape[0]
    S = sT.shape[1]
    Qb = min(Qb, Q)
    grid = (Q // Qb,)
    return pl.pallas_call(
        _knn_body,
        grid=grid,
        in_specs=[
            pl.BlockSpec((Qb, 8), lambda i: (i, 0)),
            pl.BlockSpec((8, S), lambda i: (0, 0)),
        ],
        out_specs=[
            pl.BlockSpec((Qb, K), lambda i: (i, 0)),
            pl.BlockSpec((Qb, K), lambda i: (i, 0)),
            pl.BlockSpec((Qb, 4), lambda i: (i, 0)),
        ],
        out_shape=[
            jax.ShapeDtypeStruct((Q, K), I32),
            jax.ShapeDtypeStruct((Q, K), F32),
            jax.ShapeDtypeStruct((Q, 4), I32),
        ],
        scratch_shapes=[pltpu.VMEM((Qb, S), F32)],
    )(qp8, sT)


def _prep_body(f_ref, p_ref, Wf_ref, Wr_ref, b_ref, o_ref, *, preact):
    f = f_ref[...]
    if preact:
        f = jnp.maximum(f, 0.0)
    o_ref[...] = (jnp.dot(f, Wf_ref[...], preferred_element_type=F32)
                  + jnp.dot(p_ref[...], Wr_ref[...], preferred_element_type=F32)
                  + b_ref[...])


def _prep(f, p8, Wf, Wr8, b, preact, Sb=512):
    S, C = f.shape
    dout = Wf.shape[1]
    if dout < 128:
        Wf = jnp.pad(Wf, ((0, 0), (0, 128 - dout)))
        Wr8 = jnp.pad(Wr8, ((0, 0), (0, 128 - dout)))
        b = jnp.pad(b, ((0, 0), (0, 128 - b.shape[1])))
        dout = 128
    Sb = min(Sb, S)
    return pl.pallas_call(
        functools.partial(_prep_body, preact=preact),
        grid=(S // Sb,),
        in_specs=[
            pl.BlockSpec((Sb, C), lambda i: (i, 0)),
            pl.BlockSpec((Sb, 8), lambda i: (i, 0)),
            pl.BlockSpec((C, dout), lambda i: (0, 0)),
            pl.BlockSpec((8, dout), lambda i: (0, 0)),
            pl.BlockSpec((1, dout), lambda i: (0, 0)),
        ],
        out_specs=pl.BlockSpec((Sb, dout), lambda i: (i, 0)),
        out_shape=jax.ShapeDtypeStruct((S, dout), F32),
    )(f, p8, Wf, Wr8, b)


def _post_core(m, qp, Wr, W2, b2):
    qg = jnp.dot(qp, Wr, preferred_element_type=F32)
    return jnp.dot(jnp.maximum(m - qg, 0.0), W2,
                   preferred_element_type=F32) + b2


def _qf_from(d_ref, nf_ref):
    d3 = d_ref[...][:, :3]
    w = 1.0 / (d3 + 1e-8)
    w = w / jnp.sum(w, axis=1, keepdims=True)
    nf = nf_ref[...]
    return (w[:, 0:1] * nf[:, 0:128] + w[:, 1:2] * nf[:, 128:256]
            + w[:, 2:3] * nf[:, 256:384])


def _post_enc_body(m_ref, qp_ref, Wr_ref, W2_ref, b2_ref, o_ref, *, dout):
    o_ref[...] = _post_core(m_ref[...][:, :dout], qp_ref[...], Wr_ref[...],
                            W2_ref[...], b2_ref[...])


def _post_enc(m, qp8, Wr8, W2, b2, Qb=512):
    Q = m.shape[0]
    dout = W2.shape[0]
    Qb = min(Qb, Q)
    return pl.pallas_call(
        functools.partial(_post_enc_body, dout=dout),
        grid=(Q // Qb,),
        in_specs=[
            pl.BlockSpec((Qb, 128), lambda i: (i, 0)),
            pl.BlockSpec((Qb, 8), lambda i: (i, 0)),
            pl.BlockSpec((8, dout), lambda i: (0, 0)),
            pl.BlockSpec((dout, dout), lambda i: (0, 0)),
            pl.BlockSpec((1, dout), lambda i: (0, 0)),
        ],
        out_specs=pl.BlockSpec((Qb, dout), lambda i: (i, 0)),
        out_shape=jax.ShapeDtypeStruct((Q, dout), F32),
    )(m, qp8, Wr8, W2, b2)


def _post_dec_body(m_ref, qp_ref, Wr_ref, W2_ref, b2_ref, d_ref, nf_ref,
                   sf_ref, lW_ref, lb_ref, o_ref):
    out = _post_core(m_ref[...], qp_ref[...], Wr_ref[...], W2_ref[...],
                     b2_ref[...])
    out = out + _qf_from(d_ref, nf_ref)
    o_ref[...] = out + jnp.dot(sf_ref[...], lW_ref[...],
                               preferred_element_type=F32) + lb_ref[...]


def _post_dec_direct_body(m_ref, qp_ref, Wr_ref, W2_ref, b2_ref, d_ref,
                          nf_ref, sk_ref, o_ref):
    out = _post_core(m_ref[...], qp_ref[...], Wr_ref[...], W2_ref[...],
                     b2_ref[...])
    o_ref[...] = out + _qf_from(d_ref, nf_ref) + sk_ref[...]


def _post_dec(m, qp8, Wr8, W2, b2, d16, nf3, skip_f, linW, linb, Qb=512):
    Q = m.shape[0]
    Qb = min(Qb, Q)
    Cs = skip_f.shape[1]
    return pl.pallas_call(
        _post_dec_body,
        grid=(Q // Qb,),
        in_specs=[
            pl.BlockSpec((Qb, 128), lambda i: (i, 0)),
            pl.BlockSpec((Qb, 8), lambda i: (i, 0)),
            pl.BlockSpec((8, 128), lambda i: (0, 0)),
            pl.BlockSpec((128, 128), lambda i: (0, 0)),
            pl.BlockSpec((1, 128), lambda i: (0, 0)),
            pl.BlockSpec((Qb, K), lambda i: (i, 0)),
            pl.BlockSpec((Qb, 512), lambda i: (i, 0)),
            pl.BlockSpec((Qb, Cs), lambda i: (i, 0)),
            pl.BlockSpec((Cs, 128), lambda i: (0, 0)),
            pl.BlockSpec((1, 128), lambda i: (0, 0)),
        ],
        out_specs=pl.BlockSpec((Qb, 128), lambda i: (i, 0)),
        out_shape=jax.ShapeDtypeStruct((Q, 128), F32),
    )(m, qp8, Wr8, W2, b2, d16, nf3, skip_f, linW, linb)


def _post_dec_direct(m, qp8, Wr8, W2, b2, d16, nf3, skip, Qb=512):
    Q = m.shape[0]
    Qb = min(Qb, Q)
    return pl.pallas_call(
        _post_dec_direct_body,
        grid=(Q // Qb,),
        in_specs=[
            pl.BlockSpec((Qb, 128), lambda i: (i, 0)),
            pl.BlockSpec((Qb, 8), lambda i: (i, 0)),
            pl.BlockSpec((8, 128), lambda i: (0, 0)),
            pl.BlockSpec((128, 128), lambda i: (0, 0)),
            pl.BlockSpec((1, 128), lambda i: (0, 0)),
            pl.BlockSpec((Qb, K), lambda i: (i, 0)),
            pl.BlockSpec((Qb, 512), lambda i: (i, 0)),
            pl.BlockSpec((Qb, 128), lambda i: (i, 0)),
        ],
        out_specs=pl.BlockSpec((Qb, 128), lambda i: (i, 0)),
        out_shape=jax.ShapeDtypeStruct((Q, 128), F32),
    )(m, qp8, Wr8, W2, b2, d16, nf3, skip)


def _mlp_body(x_ref, w0, b0, w1, b1, w2, b2, w3, b3, o_ref):
    h = jnp.maximum(jnp.dot(x_ref[...], w0[...], preferred_element_type=F32)
                    + b0[...], 0.0)
    h = jnp.maximum(jnp.dot(h, w1[...], preferred_element_type=F32)
                    + b1[...], 0.0)
    h = jnp.maximum(jnp.dot(h, w2[...], preferred_element_type=F32)
                    + b2[...], 0.0)
    o_ref[...] = jnp.dot(h, w3[...], preferred_element_type=F32) + b3[...]


def _mlp(x, ws, Qb=1024):
    Q = x.shape[0]
    w0, b0, w1, b1, w2, b2, w3, b3 = ws
    specs = [pl.BlockSpec((Qb, 128), lambda i: (i, 0))]
    for wt, bt in ((w0, b0), (w1, b1), (w2, b2), (w3, b3)):
        specs.append(pl.BlockSpec(wt.shape, lambda i: (0, 0)))
        specs.append(pl.BlockSpec((1, bt.shape[1]), lambda i: (0, 0)))
    return pl.pallas_call(
        _mlp_body,
        grid=(Q // Qb,),
        in_specs=specs,
        out_specs=pl.BlockSpec((Qb, 3), lambda i: (i, 0)),
        out_shape=jax.ShapeDtypeStruct((Q, 3), F32),
    )(x, w0, b0, w1, b1, w2, b2, w3, b3)


def _sc_chunk_max(rows_v, mbuf, cq, D):
    def qbody(q, _):
        for c in range(D // 16):
            acc = rows_v[q * K, pl.ds(c * 16, 16)]
            for k in range(1, K):
                acc = jnp.maximum(acc, rows_v[q * K + k, pl.ds(c * 16, 16)])
            mbuf[q, pl.ds(c * 16, 16)] = acc
        return 0
    lax.fori_loop(0, cq, qbody, 0, unroll=False)


def _sc_gather_max_enc(table, idx_flat, D):
    Qt = idx_flat.shape[0]
    Q = Qt // K
    nq = Q // NW
    cq = min(nq, 16)
    nchunks = nq // cq
    mesh = plsc.VectorSubcoreMesh(core_axis_name="c", subcore_axis_name="s")

    @functools.partial(
        pl.kernel, mesh=mesh,
        out_type=jax.ShapeDtypeStruct((Q, D), F32),
        scratch_types=[
            pltpu.VMEM((cq * K,), I32),
            pltpu.VMEM((cq * K, D), F32),
            pltpu.VMEM((cq, D), F32),
            pltpu.SemaphoreType.DMA,
        ],
    )
    def k(table_hbm, idx_hbm, m_hbm, idx_v, rows_v, mbuf, sem):
        wid = lax.axis_index("s") * 2 + lax.axis_index("c")

        def chunk(ch, _):
            base = wid * nq + ch * cq
            pltpu.sync_copy(idx_hbm.at[pl.ds(base * K, cq * K)], idx_v)
            pltpu.async_copy(table_hbm.at[idx_v], rows_v, sem).wait()
            _sc_chunk_max(rows_v, mbuf, cq, D)
            pltpu.sync_copy(mbuf, m_hbm.at[pl.ds(base, cq)])
            return 0

        lax.fori_loop(0, nchunks, chunk, 0, unroll=False)

    return k(table, idx_flat)


def _sc_gather_max_dec(table, table3, idx_flat, idx4_flat, D):
    Qt = idx_flat.shape[0]
    Q = Qt // K
    nq = Q // NW
    cq = min(nq, 16)
    nchunks = nq // cq
    mesh = plsc.VectorSubcoreMesh(core_axis_name="c", subcore_axis_name="s")

    @functools.partial(
        pl.kernel, mesh=mesh,
        out_type=(jax.ShapeDtypeStruct((Q, D), F32),
                  jax.ShapeDtypeStruct((Q * 4, D), F32)),
        scratch_types=[
            pltpu.VMEM((cq * K,), I32),
            pltpu.VMEM((cq * K, D), F32),
            pltpu.VMEM((cq, D), F32),
            pltpu.VMEM((cq * 4,), I32),
            pltpu.VMEM((cq * 4, D), F32),
            pltpu.SemaphoreType.DMA,
            pltpu.SemaphoreType.DMA,
        ],
    )
    def k(table_hbm, table3_hbm, idx_hbm, idx4_hbm, m_hbm, nf4_hbm,
          idx_v, rows_v, mbuf, idx4_v, rows4_v, sem, sem4):
        wid = lax.axis_index("s") * 2 + lax.axis_index("c")

        def chunk(ch, _):
            base = wid * nq + ch * cq
            pltpu.sync_copy(idx_hbm.at[pl.ds(base * K, cq * K)], idx_v)
            pltpu.sync_copy(idx4_hbm.at[pl.ds(base * 4, cq * 4)], idx4_v)
            pltpu.async_copy(table_hbm.at[idx_v], rows_v, sem).wait()
            pltpu.async_copy(table3_hbm.at[idx4_v], rows4_v, sem4).wait()
            _sc_chunk_max(rows_v, mbuf, cq, D)
            pltpu.sync_copy(mbuf, m_hbm.at[pl.ds(base, cq)])
            pltpu.sync_copy(rows4_v, nf4_hbm.at[pl.ds(base * 4, cq * 4)])
            return 0

        lax.fori_loop(0, nchunks, chunk, 0, unroll=False)

    return k(table, table3, idx_flat, idx4_flat)


def _pad8(x):
    return jnp.pad(x, ((0, 0), (0, 8 - x.shape[1])))


def _split_W1(W1, C):
    Wf = W1[:C]
    if C == 6:
        Wf = jnp.pad(Wf, ((0, 2), (0, 0)))
    Wr8 = jnp.pad(W1[C:], ((0, 5), (0, 0)))
    return Wf, Wr8


def kernel(points, features, enc0_W1, enc0_b1, enc0_W2, enc0_b2, enc1_W1, enc1_b1, enc1_W2, enc1_b2, enc2_W1, enc2_b1, enc2_W2, enc2_b2, enc3_W1, enc3_b1, enc3_W2, enc3_b2, up0_W1, up0_b1, up0_W2, up0_b2, up1_W1, up1_b1, up1_W2, up1_b2, up2_W1, up2_b1, up2_W2, up2_b2, up3_W1, up3_b1, up3_W2, up3_b2, lin0_W, lin0_b, lin1_W, lin1_b, lin2_W, lin2_b, mlp0_W, mlp0_b, mlp1_W, mlp1_b, mlp2_W, mlp2_b, mlp3_W, mlp3_b):
    r1 = lambda b: b.reshape(1, -1)
    pts8 = _pad8(points)
    feat8 = _pad8(features)
    q0 = pts8[::4]
    q1 = q0[::4]
    q2 = q1[::4]
    ptsT = pts8.T
    q0T = q0.T
    q1T = q1.T
    q2T = q2.T

    i0, _, _ = _knn(q0, ptsT)
    i1, _, _ = _knn(q1, q0T)
    i2, _, _ = _knn(q2, q1T)
    i3, d3, i3_4 = _knn(q2, q2T)
    iu2, du2, iu2_4 = _knn(q1, q2T)
    iu1, du1, iu1_4 = _knn(q0, q1T)
    iu0, du0, iu0_4 = _knn(pts8, q0T)

    Wf, Wr = _split_W1(enc0_W1, 6)
    c = _prep(feat8, pts8, jnp.pad(enc0_W1[:6], ((0, 2), (0, 0))), Wr,
              r1(enc0_b1), False)
    m = _sc_gather_max_enc(c, i0.reshape(-1), 128)
    f0 = _post_enc(m, q0, Wr, enc0_W2, r1(enc0_b2))

    Wf, Wr = _split_W1(enc1_W1, 64)
    c = _prep(f0, q0, Wf, Wr, r1(enc1_b1), True)
    m = _sc_gather_max_enc(c, i1.reshape(-1), 128)
    f1 = _post_enc(m, q1, Wr, enc1_W2, r1(enc1_b2))

    Wf, Wr = _split_W1(enc2_W1, 96)
    c = _prep(f1, q1, Wf, Wr, r1(enc2_b1), True)
    m = _sc_gather_max_enc(c, i2.reshape(-1), 128)
    f2 = _post_enc(m, q2, Wr, enc2_W2, r1(enc2_b2))

    Wf, Wr = _split_W1(enc3_W1, 128)
    c = _prep(f2, q2, Wf, Wr, r1(enc3_b1), True)
    m = _sc_gather_max_enc(c, i3.reshape(-1), 128)
    f3 = _post_enc(m, q2, Wr, enc3_W2, r1(enc3_b2))

    Wf, Wr = _split_W1(up3_W1, 128)
    c = _prep(f3, q2, Wf, Wr, r1(up3_b1), True)
    m, nf4 = _sc_gather_max_dec(c, f3, i3.reshape(-1), i3_4.reshape(-1), 128)
    fe = _post_dec_direct(m, q2, Wr, up3_W2, r1(up3_b2), d3,
                          nf4.reshape(-1, 512), f2)

    Wf, Wr = _split_W1(up2_W1, 128)
    c = _prep(fe, q2, Wf, Wr, r1(up2_b1), True)
    m, nf4 = _sc_gather_max_dec(c, fe, iu2.reshape(-1), iu2_4.reshape(-1), 128)
    fe = _post_dec(m, q1, Wr, up2_W2, r1(up2_b2), du2,
                   nf4.reshape(-1, 512), f1, lin2_W, r1(lin2_b))

    Wf, Wr = _split_W1(up1_W1, 128)
    c = _prep(fe, q1, Wf, Wr, r1(up1_b1), True)
    m, nf4 = _sc_gather_max_dec(c, fe, iu1.reshape(-1), iu1_4.reshape(-1), 128)
    fe = _post_dec(m, q0, Wr, up1_W2, r1(up1_b2), du1,
                   nf4.reshape(-1, 512), f0, lin1_W, r1(lin1_b))

    Wf, Wr = _split_W1(up0_W1, 128)
    c = _prep(fe, q0, Wf, Wr, r1(up0_b1), True)
    m, nf4 = _sc_gather_max_dec(c, fe, iu0.reshape(-1), iu0_4.reshape(-1), 128)
    fe = _post_dec(m, pts8, Wr, up0_W2, r1(up0_b2), du0,
                   nf4.reshape(-1, 512), feat8,
                   jnp.pad(lin0_W, ((0, 2), (0, 0))), r1(lin0_b))

    return _mlp(fe, (mlp0_W, r1(mlp0_b), mlp1_W, r1(mlp1_b),
                     mlp2_W, r1(mlp2_b), mlp3_W, r1(mlp3_b)))


# speedup vs baseline: 8.5428x; 1.1380x over previous
import functools

import jax
import jax.numpy as jnp
from jax import lax
from jax.experimental import pallas as pl
from jax.experimental.pallas import tpu as pltpu
from jax.experimental.pallas import tpu_sc as plsc

F32 = jnp.float32
I32 = jnp.int32
K = 16
NW = 32


def _write_topk(vals_iter, idx_ref, d_ref, i4_ref, get_mn_am):
    idx_cols, d_cols = [], []
    for _ in range(K):
        mn, am = get_mn_am()
        idx_cols.append(am)
        d_cols.append(jnp.maximum(mn, 0.0))
    idx_ref[...] = jnp.concatenate(idx_cols, axis=1)
    d_ref[...] = jnp.concatenate(d_cols, axis=1)
    i4_ref[...] = jnp.concatenate(idx_cols[:4], axis=1)


def _slow_extract(D_ref, idx_ref, d_ref, i4_ref, Qb, S):
    cols = lax.broadcasted_iota(I32, (Qb, S), 1)

    def step():
        Dk = D_ref[...]
        mn = jnp.min(Dk, axis=1, keepdims=True)
        am = jnp.min(jnp.where(Dk <= mn, cols, S), axis=1, keepdims=True)
        D_ref[...] = jnp.where(cols == am, jnp.inf, Dk)
        return mn, am

    _write_topk(None, idx_ref, d_ref, i4_ref, step)


def _knn_body(qp_ref, sT_ref, idx_ref, d_ref, i4_ref, D_ref):
    q = qp_ref[...]
    sT = sT_ref[...]
    qsq = jnp.sum(q * q, axis=1, keepdims=True)
    ssq = jnp.sum(sT * sT, axis=0, keepdims=True)
    D = qsq + ssq - 2.0 * jnp.dot(q, sT, preferred_element_type=F32)
    D_ref[...] = D
    S = sT.shape[1]
    Qb = q.shape[0]
    if S < 2048:
        _slow_extract(D_ref, idx_ref, d_ref, i4_ref, Qb, S)
        return
    NC = 5
    G = S // 128
    vs = [jnp.full((Qb, 128), jnp.inf, F32) for _ in range(NC)]
    rs = [jnp.zeros((Qb, 128), I32) for _ in range(NC)]
    for g in range(G):
        x = D_ref[:, g * 128:(g + 1) * 128]
        xr = jnp.full((Qb, 128), g, I32)
        for i in range(NC):
            c = x < vs[i]
            nv = jnp.where(c, x, vs[i])
            nr = jnp.where(c, xr, rs[i])
            x = jnp.where(c, vs[i], x)
            xr = jnp.where(c, rs[i], xr)
            vs[i] = nv
            rs[i] = nr
    lane = lax.broadcasted_iota(I32, (Qb, 128), 1)
    Cw = [jnp.concatenate(vs, axis=1)]
    CI = jnp.concatenate([r * 128 + lane for r in rs], axis=1)

    def step():
        mn = jnp.min(Cw[0], axis=1, keepdims=True)
        am = jnp.min(jnp.where(Cw[0] <= mn, CI, S), axis=1, keepdims=True)
        Cw[0] = jnp.where(CI == am, jnp.inf, Cw[0])
        return mn, am

    _write_topk(None, idx_ref, d_ref, i4_ref, step)
    consumed_last = Cw[0][:, (NC - 1) * 128:] == jnp.inf
    flag = jnp.any(consumed_last)

    @pl.when(flag)
    def _():
        _slow_extract(D_ref, idx_ref, d_ref, i4_ref, Qb, S)


def _knn(qp8, sT, Qb=256):
    Q = qp8.shape[0]
    S = sT.shape[1]
    Qb = min(Qb, Q)
    grid = (Q // Qb,)
    return pl.pallas_call(
        _knn_body,
        grid=grid,
        in_specs=[
            pl.BlockSpec((Qb, 8), lambda i: (i, 0)),
            pl.BlockSpec((8, S), lambda i: (0, 0)),
        ],
        out_specs=[
            pl.BlockSpec((Qb, K), lambda i: (i, 0)),
            pl.BlockSpec((Qb, K), lambda i: (i, 0)),
            pl.BlockSpec((Qb, 4), lambda i: (i, 0)),
        ],
        out_shape=[
            jax.ShapeDtypeStruct((Q, K), I32),
            jax.ShapeDtypeStruct((Q, K), F32),
            jax.ShapeDtypeStruct((Q, 4), I32),
        ],
        scratch_shapes=[pltpu.VMEM((Qb, S), F32)],
    )(qp8, sT)


def _prep_body(f_ref, p_ref, Wf_ref, Wr_ref, b_ref, o_ref, *, preact):
    f = f_ref[...]
    if preact:
        f = jnp.maximum(f, 0.0)
    o_ref[...] = (jnp.dot(f, Wf_ref[...], preferred_element_type=F32)
                  + jnp.dot(p_ref[...], Wr_ref[...], preferred_element_type=F32)
                  + b_ref[...])


def _prep(f, p8, Wf, Wr8, b, preact, Sb=512):
    S, C = f.shape
    dout = Wf.shape[1]
    if dout < 128:
        Wf = jnp.pad(Wf, ((0, 0), (0, 128 - dout)))
        Wr8 = jnp.pad(Wr8, ((0, 0), (0, 128 - dout)))
        b = jnp.pad(b, ((0, 0), (0, 128 - b.shape[1])))
        dout = 128
    Sb = min(Sb, S)
    return pl.pallas_call(
        functools.partial(_prep_body, preact=preact),
        grid=(S // Sb,),
        in_specs=[
            pl.BlockSpec((Sb, C), lambda i: (i, 0)),
            pl.BlockSpec((Sb, 8), lambda i: (i, 0)),
            pl.BlockSpec((C, dout), lambda i: (0, 0)),
            pl.BlockSpec((8, dout), lambda i: (0, 0)),
            pl.BlockSpec((1, dout), lambda i: (0, 0)),
        ],
        out_specs=pl.BlockSpec((Sb, dout), lambda i: (i, 0)),
        out_shape=jax.ShapeDtypeStruct((S, dout), F32),
    )(f, p8, Wf, Wr8, b)


def _post_core(m, qp, Wr, W2, b2):
    qg = jnp.dot(qp, Wr, preferred_element_type=F32)
    return jnp.dot(jnp.maximum(m - qg, 0.0), W2,
                   preferred_element_type=F32) + b2


def _qf_from(d_ref, nf_ref):
    d3 = d_ref[...][:, :3]
    w = 1.0 / (d3 + 1e-8)
    w = w / jnp.sum(w, axis=1, keepdims=True)
    nf = nf_ref[...]
    return (w[:, 0:1] * nf[:, 0:128] + w[:, 1:2] * nf[:, 128:256]
            + w[:, 2:3] * nf[:, 256:384])


def _post_enc_body(m_ref, qp_ref, Wr_ref, W2_ref, b2_ref, o_ref, *, dout):
    o_ref[...] = _post_core(m_ref[...][:, :dout], qp_ref[...], Wr_ref[...],
                            W2_ref[...], b2_ref[...])


def _post_enc(m, qp8, Wr8, W2, b2, Qb=512):
    Q = m.shape[0]
    dout = W2.shape[0]
    Qb = min(Qb, Q)
    return pl.pallas_call(
        functools.partial(_post_enc_body, dout=dout),
        grid=(Q // Qb,),
        in_specs=[
            pl.BlockSpec((Qb, 128), lambda i: (i, 0)),
            pl.BlockSpec((Qb, 8), lambda i: (i, 0)),
            pl.BlockSpec((8, dout), lambda i: (0, 0)),
            pl.BlockSpec((dout, dout), lambda i: (0, 0)),
            pl.BlockSpec((1, dout), lambda i: (0, 0)),
        ],
        out_specs=pl.BlockSpec((Qb, dout), lambda i: (i, 0)),
        out_shape=jax.ShapeDtypeStruct((Q, dout), F32),
    )(m, qp8, Wr8, W2, b2)


def _post_dec_body(m_ref, qp_ref, Wr_ref, W2_ref, b2_ref, d_ref, nf_ref,
                   sf_ref, lW_ref, lb_ref, o_ref):
    out = _post_core(m_ref[...], qp_ref[...], Wr_ref[...], W2_ref[...],
                     b2_ref[...])
    out = out + _qf_from(d_ref, nf_ref)
    o_ref[...] = out + jnp.dot(sf_ref[...], lW_ref[...],
                               preferred_element_type=F32) + lb_ref[...]


def _post_dec_direct_body(m_ref, qp_ref, Wr_ref, W2_ref, b2_ref, d_ref,
                          nf_ref, sk_ref, o_ref):
    out = _post_core(m_ref[...], qp_ref[...], Wr_ref[...], W2_ref[...],
                     b2_ref[...])
    o_ref[...] = out + _qf_from(d_ref, nf_ref) + sk_ref[...]


def _post_dec(m, qp8, Wr8, W2, b2, d16, nf3, skip_f, linW, linb, Qb=512):
    Q = m.shape[0]
    Qb = min(Qb, Q)
    Cs = skip_f.shape[1]
    return pl.pallas_call(
        _post_dec_body,
        grid=(Q // Qb,),
        in_specs=[
            pl.BlockSpec((Qb, 128), lambda i: (i, 0)),
            pl.BlockSpec((Qb, 8), lambda i: (i, 0)),
            pl.BlockSpec((8, 128), lambda i: (0, 0)),
            pl.BlockSpec((128, 128), lambda i: (0, 0)),
            pl.BlockSpec((1, 128), lambda i: (0, 0)),
            pl.BlockSpec((Qb, K), lambda i: (i, 0)),
            pl.BlockSpec((Qb, 512), lambda i: (i, 0)),
            pl.BlockSpec((Qb, Cs), lambda i: (i, 0)),
            pl.BlockSpec((Cs, 128), lambda i: (0, 0)),
            pl.BlockSpec((1, 128), lambda i: (0, 0)),
        ],
        out_specs=pl.BlockSpec((Qb, 128), lambda i: (i, 0)),
        out_shape=jax.ShapeDtypeStruct((Q, 128), F32),
    )(m, qp8, Wr8, W2, b2, d16, nf3, skip_f, linW, linb)


def _post_dec_direct(m, qp8, Wr8, W2, b2, d16, nf3, skip, Qb=512):
    Q = m.shape[0]
    Qb = min(Qb, Q)
    return pl.pallas_call(
        _post_dec_direct_body,
        grid=(Q // Qb,),
        in_specs=[
            pl.BlockSpec((Qb, 128), lambda i: (i, 0)),
            pl.BlockSpec((Qb, 8), lambda i: (i, 0)),
            pl.BlockSpec((8, 128), lambda i: (0, 0)),
            pl.BlockSpec((128, 128), lambda i: (0, 0)),
            pl.BlockSpec((1, 128), lambda i: (0, 0)),
            pl.BlockSpec((Qb, K), lambda i: (i, 0)),
            pl.BlockSpec((Qb, 512), lambda i: (i, 0)),
            pl.BlockSpec((Qb, 128), lambda i: (i, 0)),
        ],
        out_specs=pl.BlockSpec((Qb, 128), lambda i: (i, 0)),
        out_shape=jax.ShapeDtypeStruct((Q, 128), F32),
    )(m, qp8, Wr8, W2, b2, d16, nf3, skip)


def _mlp_body(x_ref, w0, b0, w1, b1, w2, b2, w3, b3, o_ref):
    h = jnp.maximum(jnp.dot(x_ref[...], w0[...], preferred_element_type=F32)
                    + b0[...], 0.0)
    h = jnp.maximum(jnp.dot(h, w1[...], preferred_element_type=F32)
                    + b1[...], 0.0)
    h = jnp.maximum(jnp.dot(h, w2[...], preferred_element_type=F32)
                    + b2[...], 0.0)
    o_ref[...] = jnp.dot(h, w3[...], preferred_element_type=F32) + b3[...]


def _mlp(x, ws, Qb=1024):
    Q = x.shape[0]
    w0, b0, w1, b1, w2, b2, w3, b3 = ws
    specs = [pl.BlockSpec((Qb, 128), lambda i: (i, 0))]
    for wt, bt in ((w0, b0), (w1, b1), (w2, b2), (w3, b3)):
        specs.append(pl.BlockSpec(wt.shape, lambda i: (0, 0)))
        specs.append(pl.BlockSpec((1, bt.shape[1]), lambda i: (0, 0)))
    return pl.pallas_call(
        _mlp_body,
        grid=(Q // Qb,),
        in_specs=specs,
        out_specs=pl.BlockSpec((Qb, 3), lambda i: (i, 0)),
        out_shape=jax.ShapeDtypeStruct((Q, 3), F32),
    )(x, w0, b0, w1, b1, w2, b2, w3, b3)


def _sc_chunk_max(rows_v, mbuf, cq, D):
    def qbody(q, _):
        for c in range(D // 16):
            acc = rows_v[q * K, pl.ds(c * 16, 16)]
            for k in range(1, K):
                acc = jnp.maximum(acc, rows_v[q * K + k, pl.ds(c * 16, 16)])
            mbuf[q, pl.ds(c * 16, 16)] = acc
        return 0
    lax.fori_loop(0, cq, qbody, 0, unroll=False)


def _sc_gather_max_enc(table, idx_flat, D):
    Qt = idx_flat.shape[0]
    Q = Qt // K
    nq = Q // NW
    cq = min(nq, 16)
    nchunks = nq // cq
    mesh = plsc.VectorSubcoreMesh(core_axis_name="c", subcore_axis_name="s")

    @functools.partial(
        pl.kernel, mesh=mesh,
        out_type=jax.ShapeDtypeStruct((Q, D), F32),
        scratch_types=[
            pltpu.VMEM((cq * K,), I32),
            pltpu.VMEM((cq * K, D), F32),
            pltpu.VMEM((cq, D), F32),
            pltpu.SemaphoreType.DMA,
        ],
    )
    def k(table_hbm, idx_hbm, m_hbm, idx_v, rows_v, mbuf, sem):
        wid = lax.axis_index("s") * 2 + lax.axis_index("c")

        def chunk(ch, _):
            base = wid * nq + ch * cq
            pltpu.sync_copy(idx_hbm.at[pl.ds(base * K, cq * K)], idx_v)
            pltpu.async_copy(table_hbm.at[idx_v], rows_v, sem).wait()
            _sc_chunk_max(rows_v, mbuf, cq, D)
            pltpu.sync_copy(mbuf, m_hbm.at[pl.ds(base, cq)])
            return 0

        lax.fori_loop(0, nchunks, chunk, 0, unroll=False)

    return k(table, idx_flat)


def _sc_gather_max_dec(table, table3, idx_flat, idx4_flat, D):
    Qt = idx_flat.shape[0]
    Q = Qt // K
    nq = Q // NW
    cq = min(nq, 16)
    nchunks = nq // cq
    mesh = plsc.VectorSubcoreMesh(core_axis_name="c", subcore_axis_name="s")

    @functools.partial(
        pl.kernel, mesh=mesh,
        out_type=(jax.ShapeDtypeStruct((Q, D), F32),
                  jax.ShapeDtypeStruct((Q * 4, D), F32)),
        scratch_types=[
            pltpu.VMEM((cq * K,), I32),
            pltpu.VMEM((cq * K, D), F32),
            pltpu.VMEM((cq, D), F32),
            pltpu.VMEM((cq * 4,), I32),
            pltpu.VMEM((cq * 4, D), F32),
            pltpu.SemaphoreType.DMA,
            pltpu.SemaphoreType.DMA,
        ],
    )
    def k(table_hbm, table3_hbm, idx_hbm, idx4_hbm, m_hbm, nf4_hbm,
          idx_v, rows_v, mbuf, idx4_v, rows4_v, sem, sem4):
        wid = lax.axis_index("s") * 2 + lax.axis_index("c")

        def chunk(ch, _):
            base = wid * nq + ch * cq
            pltpu.sync_copy(idx_hbm.at[pl.ds(base * K, cq * K)], idx_v)
            pltpu.sync_copy(idx4_hbm.at[pl.ds(base * 4, cq * 4)], idx4_v)
            pltpu.async_copy(table_hbm.at[idx_v], rows_v, sem).wait()
            pltpu.async_copy(table3_hbm.at[idx4_v], rows4_v, sem4).wait()
            _sc_chunk_max(rows_v, mbuf, cq, D)
            pltpu.sync_copy(mbuf, m_hbm.at[pl.ds(base, cq)])
            pltpu.sync_copy(rows4_v, nf4_hbm.at[pl.ds(base * 4, cq * 4)])
            return 0

        lax.fori_loop(0, nchunks, chunk, 0, unroll=False)

    return k(table, table3, idx_flat, idx4_flat)


def _pad8(x):
    return jnp.pad(x, ((0, 0), (0, 8 - x.shape[1])))


def _split_W1(W1, C):
    Wf = W1[:C]
    if C == 6:
        Wf = jnp.pad(Wf, ((0, 2), (0, 0)))
    Wr8 = jnp.pad(W1[C:], ((0, 5), (0, 0)))
    return Wf, Wr8


def kernel(points, features, enc0_W1, enc0_b1, enc0_W2, enc0_b2, enc1_W1, enc1_b1, enc1_W2, enc1_b2, enc2_W1, enc2_b1, enc2_W2, enc2_b2, enc3_W1, enc3_b1, enc3_W2, enc3_b2, up0_W1, up0_b1, up0_W2, up0_b2, up1_W1, up1_b1, up1_W2, up1_b2, up2_W1, up2_b1, up2_W2, up2_b2, up3_W1, up3_b1, up3_W2, up3_b2, lin0_W, lin0_b, lin1_W, lin1_b, lin2_W, lin2_b, mlp0_W, mlp0_b, mlp1_W, mlp1_b, mlp2_W, mlp2_b, mlp3_W, mlp3_b):
    r1 = lambda b: b.reshape(1, -1)
    pts8 = _pad8(points)
    feat8 = _pad8(features)
    q0 = pts8[::4]
    q1 = q0[::4]
    q2 = q1[::4]
    ptsT = pts8.T
    q0T = q0.T
    q1T = q1.T
    q2T = q2.T

    i0, _, _ = _knn(q0, ptsT)
    i1, _, _ = _knn(q1, q0T)
    i2, _, _ = _knn(q2, q1T)
    i3, d3, i3_4 = _knn(q2, q2T)
    iu2, du2, iu2_4 = _knn(q1, q2T)
    iu1, du1, iu1_4 = _knn(q0, q1T)
    iu0, du0, iu0_4 = _knn(pts8, q0T)

    Wf, Wr = _split_W1(enc0_W1, 6)
    c = _prep(feat8, pts8, jnp.pad(enc0_W1[:6], ((0, 2), (0, 0))), Wr,
              r1(enc0_b1), False)
    m = _sc_gather_max_enc(c, i0.reshape(-1), 128)
    f0 = _post_enc(m, q0, Wr, enc0_W2, r1(enc0_b2))

    Wf, Wr = _split_W1(enc1_W1, 64)
    c = _prep(f0, q0, Wf, Wr, r1(enc1_b1), True)
    m = _sc_gather_max_enc(c, i1.reshape(-1), 128)
    f1 = _post_enc(m, q1, Wr, enc1_W2, r1(enc1_b2))

    Wf, Wr = _split_W1(enc2_W1, 96)
    c = _prep(f1, q1, Wf, Wr, r1(enc2_b1), True)
    m = _sc_gather_max_enc(c, i2.reshape(-1), 128)
    f2 = _post_enc(m, q2, Wr, enc2_W2, r1(enc2_b2))

    Wf, Wr = _split_W1(enc3_W1, 128)
    c = _prep(f2, q2, Wf, Wr, r1(enc3_b1), True)
    m = _sc_gather_max_enc(c, i3.reshape(-1), 128)
    f3 = _post_enc(m, q2, Wr, enc3_W2, r1(enc3_b2))

    Wf, Wr = _split_W1(up3_W1, 128)
    c = _prep(f3, q2, Wf, Wr, r1(up3_b1), True)
    m, nf4 = _sc_gather_max_dec(c, f3, i3.reshape(-1), i3_4.reshape(-1), 128)
    fe = _post_dec_direct(m, q2, Wr, up3_W2, r1(up3_b2), d3,
                          nf4.reshape(-1, 512), f2)

    Wf, Wr = _split_W1(up2_W1, 128)
    c = _prep(fe, q2, Wf, Wr, r1(up2_b1), True)
    m, nf4 = _sc_gather_max_dec(c, fe, iu2.reshape(-1), iu2_4.reshape(-1), 128)
    fe = _post_dec(m, q1, Wr, up2_W2, r1(up2_b2), du2,
                   nf4.reshape(-1, 512), f1, lin2_W, r1(lin2_b))

    Wf, Wr = _split_W1(up1_W1, 128)
    c = _prep(fe, q1, Wf, Wr, r1(up1_b1), True)
    m, nf4 = _sc_gather_max_dec(c, fe, iu1.reshape(-1), iu1_4.reshape(-1), 128)
    fe = _post_dec(m, q0, Wr, up1_W2, r1(up1_b2), du1,
                   nf4.reshape(-1, 512), f0, lin1_W, r1(lin1_b))

    Wf, Wr = _split_W1(up0_W1, 128)
    c = _prep(fe, q0, Wf, Wr, r1(up0_b1), True)
    m, nf4 = _sc_gather_max_dec(c, fe, iu0.reshape(-1), iu0_4.reshape(-1), 128)
    fe = _post_dec(m, pts8, Wr, up0_W2, r1(up0_b2), du0,
                   nf4.reshape(-1, 512), feat8,
                   jnp.pad(lin0_W, ((0, 2), (0, 0))), r1(lin0_b))

    return _mlp(fe, (mlp0_W, r1(mlp0_b), mlp1_W, r1(mlp1_b),
                     mlp2_W, r1(mlp2_b), mlp3_W, r1(mlp3_b)))


# speedup vs baseline: 8.6929x; 1.0176x over previous
import functools

import jax
import jax.numpy as jnp
from jax import lax
from jax.experimental import pallas as pl
from jax.experimental.pallas import tpu as pltpu
from jax.experimental.pallas import tpu_sc as plsc

F32 = jnp.float32
I32 = jnp.int32
K = 16
NW = 32


def _write_topk(vals_iter, idx_ref, d_ref, i4_ref, get_mn_am):
    idx_cols, d_cols = [], []
    for _ in range(K):
        mn, am = get_mn_am()
        idx_cols.append(am)
        d_cols.append(jnp.maximum(mn, 0.0))
    idx_ref[...] = jnp.concatenate(idx_cols, axis=1)
    d_ref[...] = jnp.concatenate(d_cols, axis=1)
    i4_ref[...] = jnp.concatenate(idx_cols[:4], axis=1)


def _slow_extract(D_ref, idx_ref, d_ref, i4_ref, Qb, S):
    cols = lax.broadcasted_iota(I32, (Qb, S), 1)

    def step():
        Dk = D_ref[...]
        mn = jnp.min(Dk, axis=1, keepdims=True)
        am = jnp.min(jnp.where(Dk <= mn, cols, S), axis=1, keepdims=True)
        D_ref[...] = jnp.where(cols == am, jnp.inf, Dk)
        return mn, am

    _write_topk(None, idx_ref, d_ref, i4_ref, step)


def _knn_body(qp_ref, sT_ref, idx_ref, d_ref, i4_ref, D_ref):
    q = qp_ref[...]
    sT = sT_ref[...]
    qsq = jnp.sum(q * q, axis=1, keepdims=True)
    ssq = jnp.sum(sT * sT, axis=0, keepdims=True)
    D = qsq + ssq - 2.0 * jnp.dot(q, sT, preferred_element_type=F32)
    D_ref[...] = D
    S = sT.shape[1]
    Qb = q.shape[0]
    if S < 2048:
        _slow_extract(D_ref, idx_ref, d_ref, i4_ref, Qb, S)
        return
    NC = 5
    G = S // 128
    vs = [jnp.full((Qb, 128), jnp.inf, F32) for _ in range(NC)]
    rs = [jnp.zeros((Qb, 128), I32) for _ in range(NC)]
    for g in range(G):
        x = D_ref[:, g * 128:(g + 1) * 128]
        xr = jnp.full((Qb, 128), g, I32)
        for i in range(NC):
            c = x < vs[i]
            nv = jnp.where(c, x, vs[i])
            nr = jnp.where(c, xr, rs[i])
            x = jnp.where(c, vs[i], x)
            xr = jnp.where(c, rs[i], xr)
            vs[i] = nv
            rs[i] = nr
    lane = lax.broadcasted_iota(I32, (Qb, 128), 1)
    Cw = [jnp.concatenate(vs, axis=1)]
    CI = jnp.concatenate([r * 128 + lane for r in rs], axis=1)

    def step():
        mn = jnp.min(Cw[0], axis=1, keepdims=True)
        am = jnp.min(jnp.where(Cw[0] <= mn, CI, S), axis=1, keepdims=True)
        Cw[0] = jnp.where(CI == am, jnp.inf, Cw[0])
        return mn, am

    _write_topk(None, idx_ref, d_ref, i4_ref, step)
    consumed_last = Cw[0][:, (NC - 1) * 128:] == jnp.inf
    flag = jnp.any(consumed_last)

    @pl.when(flag)
    def _():
        _slow_extract(D_ref, idx_ref, d_ref, i4_ref, Qb, S)


def _knn(qp8, sT, Qb=256):
    Q = qp8.shape[0]
    S = sT.shape[1]
    Qb = min(Qb, Q)
    grid = (Q // Qb,)
    return pl.pallas_call(
        _knn_body,
        grid=grid,
        in_specs=[
            pl.BlockSpec((Qb, 8), lambda i: (i, 0)),
            pl.BlockSpec((8, S), lambda i: (0, 0)),
        ],
        out_specs=[
            pl.BlockSpec((Qb, K), lambda i: (i, 0)),
            pl.BlockSpec((Qb, K), lambda i: (i, 0)),
            pl.BlockSpec((Qb, 4), lambda i: (i, 0)),
        ],
        out_shape=[
            jax.ShapeDtypeStruct((Q, K), I32),
            jax.ShapeDtypeStruct((Q, K), F32),
            jax.ShapeDtypeStruct((Q, 4), I32),
        ],
        scratch_shapes=[pltpu.VMEM((Qb, S), F32)],
    )(qp8, sT)


def _prep_body(f_ref, p_ref, Wf_ref, Wr_ref, b_ref, o_ref, *, preact):
    f = f_ref[...]
    if preact:
        f = jnp.maximum(f, 0.0)
    o_ref[...] = (jnp.dot(f, Wf_ref[...], preferred_element_type=F32)
                  + jnp.dot(p_ref[...], Wr_ref[...], preferred_element_type=F32)
                  + b_ref[...])


def _prep(f, p8, Wf, Wr8, b, preact, Sb=512):
    S, C = f.shape
    dout = Wf.shape[1]
    if dout < 128:
        Wf = jnp.pad(Wf, ((0, 0), (0, 128 - dout)))
        Wr8 = jnp.pad(Wr8, ((0, 0), (0, 128 - dout)))
        b = jnp.pad(b, ((0, 0), (0, 128 - b.shape[1])))
        dout = 128
    Sb = min(Sb, S)
    return pl.pallas_call(
        functools.partial(_prep_body, preact=preact),
        grid=(S // Sb,),
        in_specs=[
            pl.BlockSpec((Sb, C), lambda i: (i, 0)),
            pl.BlockSpec((Sb, 8), lambda i: (i, 0)),
            pl.BlockSpec((C, dout), lambda i: (0, 0)),
            pl.BlockSpec((8, dout), lambda i: (0, 0)),
            pl.BlockSpec((1, dout), lambda i: (0, 0)),
        ],
        out_specs=pl.BlockSpec((Sb, dout), lambda i: (i, 0)),
        out_shape=jax.ShapeDtypeStruct((S, dout), F32),
    )(f, p8, Wf, Wr8, b)


def _post_core(m, qp, Wr, W2, b2):
    qg = jnp.dot(qp, Wr, preferred_element_type=F32)
    return jnp.dot(jnp.maximum(m - qg, 0.0), W2,
                   preferred_element_type=F32) + b2


def _qf_from(d_ref, nf_ref):
    d3 = d_ref[...][:, :3]
    w = 1.0 / (d3 + 1e-8)
    w = w / jnp.sum(w, axis=1, keepdims=True)
    nf = nf_ref[...]
    return (w[:, 0:1] * nf[:, 0:128] + w[:, 1:2] * nf[:, 128:256]
            + w[:, 2:3] * nf[:, 256:384])


def _post_enc_body(m_ref, qp_ref, Wr_ref, W2_ref, b2_ref, o_ref, *, dout):
    o_ref[...] = _post_core(m_ref[...][:, :dout], qp_ref[...], Wr_ref[...],
                            W2_ref[...], b2_ref[...])


def _post_enc(m, qp8, Wr8, W2, b2, Qb=512):
    Q = m.shape[0]
    dout = W2.shape[0]
    Qb = min(Qb, Q)
    return pl.pallas_call(
        functools.partial(_post_enc_body, dout=dout),
        grid=(Q // Qb,),
        in_specs=[
            pl.BlockSpec((Qb, 128), lambda i: (i, 0)),
            pl.BlockSpec((Qb, 8), lambda i: (i, 0)),
            pl.BlockSpec((8, dout), lambda i: (0, 0)),
            pl.BlockSpec((dout, dout), lambda i: (0, 0)),
            pl.BlockSpec((1, dout), lambda i: (0, 0)),
        ],
        out_specs=pl.BlockSpec((Qb, dout), lambda i: (i, 0)),
        out_shape=jax.ShapeDtypeStruct((Q, dout), F32),
    )(m, qp8, Wr8, W2, b2)


def _post_dec_body(m_ref, qp_ref, Wr_ref, W2_ref, b2_ref, d_ref, nf_ref,
                   sf_ref, lW_ref, lb_ref, o_ref):
    out = _post_core(m_ref[...], qp_ref[...], Wr_ref[...], W2_ref[...],
                     b2_ref[...])
    out = out + _qf_from(d_ref, nf_ref)
    o_ref[...] = out + jnp.dot(sf_ref[...], lW_ref[...],
                               preferred_element_type=F32) + lb_ref[...]


def _post_dec_direct_body(m_ref, qp_ref, Wr_ref, W2_ref, b2_ref, d_ref,
                          nf_ref, sk_ref, o_ref):
    out = _post_core(m_ref[...], qp_ref[...], Wr_ref[...], W2_ref[...],
                     b2_ref[...])
    o_ref[...] = out + _qf_from(d_ref, nf_ref) + sk_ref[...]


def _post_dec_mlp_body(m_ref, qp_ref, Wr_ref, W2_ref, b2_ref, d_ref, nf_ref,
                       sf_ref, lW_ref, lb_ref,
                       w0, c0, w1, c1, w2, c2, w3, c3, o_ref):
    fe = _post_core(m_ref[...], qp_ref[...], Wr_ref[...], W2_ref[...],
                    b2_ref[...])
    fe = fe + _qf_from(d_ref, nf_ref)
    fe = fe + jnp.dot(sf_ref[...], lW_ref[...],
                      preferred_element_type=F32) + lb_ref[...]
    h = jnp.maximum(jnp.dot(fe, w0[...], preferred_element_type=F32)
                    + c0[...], 0.0)
    h = jnp.maximum(jnp.dot(h, w1[...], preferred_element_type=F32)
                    + c1[...], 0.0)
    h = jnp.maximum(jnp.dot(h, w2[...], preferred_element_type=F32)
                    + c2[...], 0.0)
    o_ref[...] = jnp.dot(h, w3[...], preferred_element_type=F32) + c3[...]


def _post_dec_mlp(m, qp8, Wr8, W2, b2, d16, nf3, skip_f, linW, linb, mlp_ws,
                  Qb=512):
    Q = m.shape[0]
    Qb = min(Qb, Q)
    Cs = skip_f.shape[1]
    specs = [
        pl.BlockSpec((Qb, 128), lambda i: (i, 0)),
        pl.BlockSpec((Qb, 8), lambda i: (i, 0)),
        pl.BlockSpec((8, 128), lambda i: (0, 0)),
        pl.BlockSpec((128, 128), lambda i: (0, 0)),
        pl.BlockSpec((1, 128), lambda i: (0, 0)),
        pl.BlockSpec((Qb, K), lambda i: (i, 0)),
        pl.BlockSpec((Qb, 512), lambda i: (i, 0)),
        pl.BlockSpec((Qb, Cs), lambda i: (i, 0)),
        pl.BlockSpec((Cs, 128), lambda i: (0, 0)),
        pl.BlockSpec((1, 128), lambda i: (0, 0)),
    ]
    for wt, bt in zip(mlp_ws[0::2], mlp_ws[1::2]):
        specs.append(pl.BlockSpec(wt.shape, lambda i: (0, 0)))
        specs.append(pl.BlockSpec((1, bt.shape[1]), lambda i: (0, 0)))
    return pl.pallas_call(
        _post_dec_mlp_body,
        grid=(Q // Qb,),
        in_specs=specs,
        out_specs=pl.BlockSpec((Qb, 3), lambda i: (i, 0)),
        out_shape=jax.ShapeDtypeStruct((Q, 3), F32),
    )(m, qp8, Wr8, W2, b2, d16, nf3, skip_f, linW, linb, *mlp_ws)


def _post_dec(m, qp8, Wr8, W2, b2, d16, nf3, skip_f, linW, linb, Qb=512):
    Q = m.shape[0]
    Qb = min(Qb, Q)
    Cs = skip_f.shape[1]
    return pl.pallas_call(
        _post_dec_body,
        grid=(Q // Qb,),
        in_specs=[
            pl.BlockSpec((Qb, 128), lambda i: (i, 0)),
            pl.BlockSpec((Qb, 8), lambda i: (i, 0)),
            pl.BlockSpec((8, 128), lambda i: (0, 0)),
            pl.BlockSpec((128, 128), lambda i: (0, 0)),
            pl.BlockSpec((1, 128), lambda i: (0, 0)),
            pl.BlockSpec((Qb, K), lambda i: (i, 0)),
            pl.BlockSpec((Qb, 512), lambda i: (i, 0)),
            pl.BlockSpec((Qb, Cs), lambda i: (i, 0)),
            pl.BlockSpec((Cs, 128), lambda i: (0, 0)),
            pl.BlockSpec((1, 128), lambda i: (0, 0)),
        ],
        out_specs=pl.BlockSpec((Qb, 128), lambda i: (i, 0)),
        out_shape=jax.ShapeDtypeStruct((Q, 128), F32),
    )(m, qp8, Wr8, W2, b2, d16, nf3, skip_f, linW, linb)


def _post_dec_direct(m, qp8, Wr8, W2, b2, d16, nf3, skip, Qb=512):
    Q = m.shape[0]
    Qb = min(Qb, Q)
    return pl.pallas_call(
        _post_dec_direct_body,
        grid=(Q // Qb,),
        in_specs=[
            pl.BlockSpec((Qb, 128), lambda i: (i, 0)),
            pl.BlockSpec((Qb, 8), lambda i: (i, 0)),
            pl.BlockSpec((8, 128), lambda i: (0, 0)),
            pl.BlockSpec((128, 128), lambda i: (0, 0)),
            pl.BlockSpec((1, 128), lambda i: (0, 0)),
            pl.BlockSpec((Qb, K), lambda i: (i, 0)),
            pl.BlockSpec((Qb, 512), lambda i: (i, 0)),
            pl.BlockSpec((Qb, 128), lambda i: (i, 0)),
        ],
        out_specs=pl.BlockSpec((Qb, 128), lambda i: (i, 0)),
        out_shape=jax.ShapeDtypeStruct((Q, 128), F32),
    )(m, qp8, Wr8, W2, b2, d16, nf3, skip)


def _mlp_body(x_ref, w0, b0, w1, b1, w2, b2, w3, b3, o_ref):
    h = jnp.maximum(jnp.dot(x_ref[...], w0[...], preferred_element_type=F32)
                    + b0[...], 0.0)
    h = jnp.maximum(jnp.dot(h, w1[...], preferred_element_type=F32)
                    + b1[...], 0.0)
    h = jnp.maximum(jnp.dot(h, w2[...], preferred_element_type=F32)
                    + b2[...], 0.0)
    o_ref[...] = jnp.dot(h, w3[...], preferred_element_type=F32) + b3[...]


def _mlp(x, ws, Qb=1024):
    Q = x.shape[0]
    w0, b0, w1, b1, w2, b2, w3, b3 = ws
    specs = [pl.BlockSpec((Qb, 128), lambda i: (i, 0))]
    for wt, bt in ((w0, b0), (w1, b1), (w2, b2), (w3, b3)):
        specs.append(pl.BlockSpec(wt.shape, lambda i: (0, 0)))
        specs.append(pl.BlockSpec((1, bt.shape[1]), lambda i: (0, 0)))
    return pl.pallas_call(
        _mlp_body,
        grid=(Q // Qb,),
        in_specs=specs,
        out_specs=pl.BlockSpec((Qb, 3), lambda i: (i, 0)),
        out_shape=jax.ShapeDtypeStruct((Q, 3), F32),
    )(x, w0, b0, w1, b1, w2, b2, w3, b3)


def _sc_chunk_max(rows_v, mbuf, cq, D):
    def qbody(q, _):
        for c in range(D // 16):
            acc = rows_v[q * K, pl.ds(c * 16, 16)]
            for k in range(1, K):
                acc = jnp.maximum(acc, rows_v[q * K + k, pl.ds(c * 16, 16)])
            mbuf[q, pl.ds(c * 16, 16)] = acc
        return 0
    lax.fori_loop(0, cq, qbody, 0, unroll=False)


def _sc_gather_max_enc(table, idx_flat, D):
    Qt = idx_flat.shape[0]
    Q = Qt // K
    nq = Q // NW
    cq = min(nq, 32)
    nchunks = nq // cq
    mesh = plsc.VectorSubcoreMesh(core_axis_name="c", subcore_axis_name="s")

    @functools.partial(
        pl.kernel, mesh=mesh,
        out_type=jax.ShapeDtypeStruct((Q, D), F32),
        scratch_types=[
            pltpu.VMEM((cq * K,), I32),
            pltpu.VMEM((cq * K, D), F32),
            pltpu.VMEM((cq, D), F32),
            pltpu.SemaphoreType.DMA,
        ],
    )
    def k(table_hbm, idx_hbm, m_hbm, idx_v, rows_v, mbuf, sem):
        wid = lax.axis_index("s") * 2 + lax.axis_index("c")

        def chunk(ch, _):
            base = wid * nq + ch * cq
            pltpu.sync_copy(idx_hbm.at[pl.ds(base * K, cq * K)], idx_v)
            pltpu.async_copy(table_hbm.at[idx_v], rows_v, sem).wait()
            _sc_chunk_max(rows_v, mbuf, cq, D)
            pltpu.sync_copy(mbuf, m_hbm.at[pl.ds(base, cq)])
            return 0

        lax.fori_loop(0, nchunks, chunk, 0, unroll=False)

    return k(table, idx_flat)


def _sc_gather_max_dec(table, table3, idx_flat, idx4_flat, D):
    Qt = idx_flat.shape[0]
    Q = Qt // K
    nq = Q // NW
    cq = min(nq, 32)
    nchunks = nq // cq
    mesh = plsc.VectorSubcoreMesh(core_axis_name="c", subcore_axis_name="s")

    @functools.partial(
        pl.kernel, mesh=mesh,
        out_type=(jax.ShapeDtypeStruct((Q, D), F32),
                  jax.ShapeDtypeStruct((Q * 4, D), F32)),
        scratch_types=[
            pltpu.VMEM((cq * K,), I32),
            pltpu.VMEM((cq * K, D), F32),
            pltpu.VMEM((cq, D), F32),
            pltpu.VMEM((cq * 4,), I32),
            pltpu.VMEM((cq * 4, D), F32),
            pltpu.SemaphoreType.DMA,
            pltpu.SemaphoreType.DMA,
        ],
    )
    def k(table_hbm, table3_hbm, idx_hbm, idx4_hbm, m_hbm, nf4_hbm,
          idx_v, rows_v, mbuf, idx4_v, rows4_v, sem, sem4):
        wid = lax.axis_index("s") * 2 + lax.axis_index("c")

        def chunk(ch, _):
            base = wid * nq + ch * cq
            pltpu.sync_copy(idx_hbm.at[pl.ds(base * K, cq * K)], idx_v)
            pltpu.sync_copy(idx4_hbm.at[pl.ds(base * 4, cq * 4)], idx4_v)
            pltpu.async_copy(table_hbm.at[idx_v], rows_v, sem).wait()
            pltpu.async_copy(table3_hbm.at[idx4_v], rows4_v, sem4).wait()
            _sc_chunk_max(rows_v, mbuf, cq, D)
            pltpu.sync_copy(mbuf, m_hbm.at[pl.ds(base, cq)])
            pltpu.sync_copy(rows4_v, nf4_hbm.at[pl.ds(base * 4, cq * 4)])
            return 0

        lax.fori_loop(0, nchunks, chunk, 0, unroll=False)

    return k(table, table3, idx_flat, idx4_flat)


def _pad8(x):
    return jnp.pad(x, ((0, 0), (0, 8 - x.shape[1])))


def _split_W1(W1, C):
    Wf = W1[:C]
    if C == 6:
        Wf = jnp.pad(Wf, ((0, 2), (0, 0)))
    Wr8 = jnp.pad(W1[C:], ((0, 5), (0, 0)))
    return Wf, Wr8


def kernel(points, features, enc0_W1, enc0_b1, enc0_W2, enc0_b2, enc1_W1, enc1_b1, enc1_W2, enc1_b2, enc2_W1, enc2_b1, enc2_W2, enc2_b2, enc3_W1, enc3_b1, enc3_W2, enc3_b2, up0_W1, up0_b1, up0_W2, up0_b2, up1_W1, up1_b1, up1_W2, up1_b2, up2_W1, up2_b1, up2_W2, up2_b2, up3_W1, up3_b1, up3_W2, up3_b2, lin0_W, lin0_b, lin1_W, lin1_b, lin2_W, lin2_b, mlp0_W, mlp0_b, mlp1_W, mlp1_b, mlp2_W, mlp2_b, mlp3_W, mlp3_b):
    r1 = lambda b: b.reshape(1, -1)
    pts8 = _pad8(points)
    feat8 = _pad8(features)
    q0 = pts8[::4]
    q1 = q0[::4]
    q2 = q1[::4]
    ptsT = pts8.T
    q0T = q0.T
    q1T = q1.T
    q2T = q2.T

    i0, _, _ = _knn(q0, ptsT)
    i1, _, _ = _knn(q1, q0T)
    i2, _, _ = _knn(q2, q1T)
    i3, d3, i3_4 = _knn(q2, q2T)
    iu2, du2, iu2_4 = _knn(q1, q2T)
    iu1, du1, iu1_4 = _knn(q0, q1T)
    iu0, du0, iu0_4 = _knn(pts8, q0T)

    Wf, Wr = _split_W1(enc0_W1, 6)
    c = _prep(feat8, pts8, jnp.pad(enc0_W1[:6], ((0, 2), (0, 0))), Wr,
              r1(enc0_b1), False)
    m = _sc_gather_max_enc(c, i0.reshape(-1), 128)
    f0 = _post_enc(m, q0, Wr, enc0_W2, r1(enc0_b2))

    Wf, Wr = _split_W1(enc1_W1, 64)
    c = _prep(f0, q0, Wf, Wr, r1(enc1_b1), True)
    m = _sc_gather_max_enc(c, i1.reshape(-1), 128)
    f1 = _post_enc(m, q1, Wr, enc1_W2, r1(enc1_b2))

    Wf, Wr = _split_W1(enc2_W1, 96)
    c = _prep(f1, q1, Wf, Wr, r1(enc2_b1), True)
    m = _sc_gather_max_enc(c, i2.reshape(-1), 128)
    f2 = _post_enc(m, q2, Wr, enc2_W2, r1(enc2_b2))

    Wf, Wr = _split_W1(enc3_W1, 128)
    c = _prep(f2, q2, Wf, Wr, r1(enc3_b1), True)
    m = _sc_gather_max_enc(c, i3.reshape(-1), 128)
    f3 = _post_enc(m, q2, Wr, enc3_W2, r1(enc3_b2))

    Wf, Wr = _split_W1(up3_W1, 128)
    c = _prep(f3, q2, Wf, Wr, r1(up3_b1), True)
    m, nf4 = _sc_gather_max_dec(c, f3, i3.reshape(-1), i3_4.reshape(-1), 128)
    fe = _post_dec_direct(m, q2, Wr, up3_W2, r1(up3_b2), d3,
                          nf4.reshape(-1, 512), f2)

    Wf, Wr = _split_W1(up2_W1, 128)
    c = _prep(fe, q2, Wf, Wr, r1(up2_b1), True)
    m, nf4 = _sc_gather_max_dec(c, fe, iu2.reshape(-1), iu2_4.reshape(-1), 128)
    fe = _post_dec(m, q1, Wr, up2_W2, r1(up2_b2), du2,
                   nf4.reshape(-1, 512), f1, lin2_W, r1(lin2_b))

    Wf, Wr = _split_W1(up1_W1, 128)
    c = _prep(fe, q1, Wf, Wr, r1(up1_b1), True)
    m, nf4 = _sc_gather_max_dec(c, fe, iu1.reshape(-1), iu1_4.reshape(-1), 128)
    fe = _post_dec(m, q0, Wr, up1_W2, r1(up1_b2), du1,
                   nf4.reshape(-1, 512), f0, lin1_W, r1(lin1_b))

    Wf, Wr = _split_W1(up0_W1, 128)
    c = _prep(fe, q0, Wf, Wr, r1(up0_b1), True)
    m, nf4 = _sc_gather_max_dec(c, fe, iu0.reshape(-1), iu0_4.reshape(-1), 128)
    return _post_dec_mlp(m, pts8, Wr, up0_W2, r1(up0_b2), du0,
                         nf4.reshape(-1, 512), feat8,
                         jnp.pad(lin0_W, ((0, 2), (0, 0))), r1(lin0_b),
                         (mlp0_W, r1(mlp0_b), mlp1_W, r1(mlp1_b),
                          mlp2_W, r1(mlp2_b), mlp3_W, r1(mlp3_b)))


# speedup vs baseline: 9.0088x; 1.0363x over previous
import functools

import jax
import jax.numpy as jnp
from jax import lax
from jax.experimental import pallas as pl
from jax.experimental.pallas import tpu as pltpu
from jax.experimental.pallas import tpu_sc as plsc

F32 = jnp.float32
I32 = jnp.int32
K = 16
NW = 32


def _write_topk(vals_iter, idx_ref, d_ref, i4_ref, get_mn_am):
    idx_cols, d_cols = [], []
    for _ in range(K):
        mn, am = get_mn_am()
        idx_cols.append(am)
        d_cols.append(jnp.maximum(mn, 0.0))
    idx_ref[...] = jnp.concatenate(idx_cols, axis=1)
    d_ref[...] = jnp.concatenate(d_cols, axis=1)
    i4_ref[...] = jnp.concatenate(idx_cols[:4], axis=1)


def _slow_extract(D_ref, idx_ref, d_ref, i4_ref, Qb, S):
    cols = lax.broadcasted_iota(I32, (Qb, S), 1)

    def step():
        Dk = D_ref[...]
        mn = jnp.min(Dk, axis=1, keepdims=True)
        am = jnp.min(jnp.where(Dk <= mn, cols, S), axis=1, keepdims=True)
        D_ref[...] = jnp.where(cols == am, jnp.inf, Dk)
        return mn, am

    _write_topk(None, idx_ref, d_ref, i4_ref, step)


def _knn_body(qp_ref, sT_ref, idx_ref, d_ref, i4_ref, D_ref):
    q = qp_ref[...]
    sT = sT_ref[...]
    qsq = jnp.sum(q * q, axis=1, keepdims=True)
    ssq = jnp.sum(sT * sT, axis=0, keepdims=True)
    D = qsq + ssq - 2.0 * jnp.dot(q, sT, preferred_element_type=F32)
    D_ref[...] = D
    S = sT.shape[1]
    Qb = q.shape[0]
    if S < 2048:
        _slow_extract(D_ref, idx_ref, d_ref, i4_ref, Qb, S)
        return
    NC = 5
    G = S // 128
    vs = [jnp.full((Qb, 128), jnp.inf, F32) for _ in range(NC)]
    rs = [jnp.zeros((Qb, 128), I32) for _ in range(NC)]
    for g in range(G):
        x = D_ref[:, g * 128:(g + 1) * 128]
        xr = jnp.full((Qb, 128), g, I32)
        for i in range(NC):
            c = x < vs[i]
            nv = jnp.where(c, x, vs[i])
            nr = jnp.where(c, xr, rs[i])
            x = jnp.where(c, vs[i], x)
            xr = jnp.where(c, rs[i], xr)
            vs[i] = nv
            rs[i] = nr
    lane = lax.broadcasted_iota(I32, (Qb, 128), 1)
    Cw = [jnp.concatenate(vs, axis=1)]
    CI = jnp.concatenate([r * 128 + lane for r in rs], axis=1)

    def step():
        mn = jnp.min(Cw[0], axis=1, keepdims=True)
        am = jnp.min(jnp.where(Cw[0] <= mn, CI, S), axis=1, keepdims=True)
        Cw[0] = jnp.where(CI == am, jnp.inf, Cw[0])
        return mn, am

    _write_topk(None, idx_ref, d_ref, i4_ref, step)
    consumed_last = Cw[0][:, (NC - 1) * 128:] == jnp.inf
    flag = jnp.any(consumed_last)

    @pl.when(flag)
    def _():
        _slow_extract(D_ref, idx_ref, d_ref, i4_ref, Qb, S)


def _knn(qp8, sT, Qb=256):
    Q = qp8.shape[0]
    S = sT.shape[1]
    Qb = min(Qb, Q)
    grid = (Q // Qb,)
    return pl.pallas_call(
        _knn_body,
        grid=grid,
        in_specs=[
            pl.BlockSpec((Qb, 8), lambda i: (i, 0)),
            pl.BlockSpec((8, S), lambda i: (0, 0)),
        ],
        out_specs=[
            pl.BlockSpec((Qb, K), lambda i: (i, 0)),
            pl.BlockSpec((Qb, K), lambda i: (i, 0)),
            pl.BlockSpec((Qb, 4), lambda i: (i, 0)),
        ],
        out_shape=[
            jax.ShapeDtypeStruct((Q, K), I32),
            jax.ShapeDtypeStruct((Q, K), F32),
            jax.ShapeDtypeStruct((Q, 4), I32),
        ],
        scratch_shapes=[pltpu.VMEM((Qb, S), F32)],
    )(qp8, sT)


def _prep_body(f_ref, p_ref, Wf_ref, Wr_ref, b_ref, o_ref, *, preact):
    f = f_ref[...]
    if preact:
        f = jnp.maximum(f, 0.0)
    o_ref[...] = (jnp.dot(f, Wf_ref[...], preferred_element_type=F32)
                  + jnp.dot(p_ref[...], Wr_ref[...], preferred_element_type=F32)
                  + b_ref[...])


def _prep(f, p8, Wf, Wr8, b, preact, Sb=512):
    S, C = f.shape
    dout = Wf.shape[1]
    if dout < 128:
        Wf = jnp.pad(Wf, ((0, 0), (0, 128 - dout)))
        Wr8 = jnp.pad(Wr8, ((0, 0), (0, 128 - dout)))
        b = jnp.pad(b, ((0, 0), (0, 128 - b.shape[1])))
        dout = 128
    Sb = min(Sb, S)
    return pl.pallas_call(
        functools.partial(_prep_body, preact=preact),
        grid=(S // Sb,),
        in_specs=[
            pl.BlockSpec((Sb, C), lambda i: (i, 0)),
            pl.BlockSpec((Sb, 8), lambda i: (i, 0)),
            pl.BlockSpec((C, dout), lambda i: (0, 0)),
            pl.BlockSpec((8, dout), lambda i: (0, 0)),
            pl.BlockSpec((1, dout), lambda i: (0, 0)),
        ],
        out_specs=pl.BlockSpec((Sb, dout), lambda i: (i, 0)),
        out_shape=jax.ShapeDtypeStruct((S, dout), F32),
    )(f, p8, Wf, Wr8, b)


def _post_core(m, qp, Wr, W2, b2):
    qg = jnp.dot(qp, Wr, preferred_element_type=F32)
    return jnp.dot(jnp.maximum(m - qg, 0.0), W2,
                   preferred_element_type=F32) + b2


def _qf_from(d_ref, nf_ref):
    d3 = d_ref[...][:, :3]
    w = 1.0 / (d3 + 1e-8)
    w = w / jnp.sum(w, axis=1, keepdims=True)
    nf = nf_ref[...]
    return (w[:, 0:1] * nf[:, 0:128] + w[:, 1:2] * nf[:, 128:256]
            + w[:, 2:3] * nf[:, 256:384])


def _post_enc_body(m_ref, qp_ref, Wr_ref, W2_ref, b2_ref, o_ref, *, dout):
    o_ref[...] = _post_core(m_ref[...][:, :dout], qp_ref[...], Wr_ref[...],
                            W2_ref[...], b2_ref[...])


def _post_enc(m, qp8, Wr8, W2, b2, Qb=512):
    Q = m.shape[0]
    dout = W2.shape[0]
    Qb = min(Qb, Q)
    return pl.pallas_call(
        functools.partial(_post_enc_body, dout=dout),
        grid=(Q // Qb,),
        in_specs=[
            pl.BlockSpec((Qb, 128), lambda i: (i, 0)),
            pl.BlockSpec((Qb, 8), lambda i: (i, 0)),
            pl.BlockSpec((8, dout), lambda i: (0, 0)),
            pl.BlockSpec((dout, dout), lambda i: (0, 0)),
            pl.BlockSpec((1, dout), lambda i: (0, 0)),
        ],
        out_specs=pl.BlockSpec((Qb, dout), lambda i: (i, 0)),
        out_shape=jax.ShapeDtypeStruct((Q, dout), F32),
    )(m, qp8, Wr8, W2, b2)


def _post_dec_body(m_ref, qp_ref, Wr_ref, W2_ref, b2_ref, d_ref, nf_ref,
                   sf_ref, lW_ref, lb_ref, o_ref):
    out = _post_core(m_ref[...], qp_ref[...], Wr_ref[...], W2_ref[...],
                     b2_ref[...])
    out = out + _qf_from(d_ref, nf_ref)
    o_ref[...] = out + jnp.dot(sf_ref[...], lW_ref[...],
                               preferred_element_type=F32) + lb_ref[...]


def _post_dec_direct_body(m_ref, qp_ref, Wr_ref, W2_ref, b2_ref, d_ref,
                          nf_ref, sk_ref, o_ref):
    out = _post_core(m_ref[...], qp_ref[...], Wr_ref[...], W2_ref[...],
                     b2_ref[...])
    o_ref[...] = out + _qf_from(d_ref, nf_ref) + sk_ref[...]


def _post_dec_mlp_body(m_ref, qp_ref, Wr_ref, W2_ref, b2_ref, d_ref, nf_ref,
                       sf_ref, lW_ref, lb_ref,
                       w0, c0, w1, c1, w2, c2, w3, c3, o_ref):
    fe = _post_core(m_ref[...], qp_ref[...], Wr_ref[...], W2_ref[...],
                    b2_ref[...])
    fe = fe + _qf_from(d_ref, nf_ref)
    fe = fe + jnp.dot(sf_ref[...], lW_ref[...],
                      preferred_element_type=F32) + lb_ref[...]
    h = jnp.maximum(jnp.dot(fe, w0[...], preferred_element_type=F32)
                    + c0[...], 0.0)
    h = jnp.maximum(jnp.dot(h, w1[...], preferred_element_type=F32)
                    + c1[...], 0.0)
    h = jnp.maximum(jnp.dot(h, w2[...], preferred_element_type=F32)
                    + c2[...], 0.0)
    o_ref[...] = jnp.dot(h, w3[...], preferred_element_type=F32) + c3[...]


def _post_dec_mlp(m, qp8, Wr8, W2, b2, d16, nf3, skip_f, linW, linb, mlp_ws,
                  Qb=512):
    Q = m.shape[0]
    Qb = min(Qb, Q)
    Cs = skip_f.shape[1]
    specs = [
        pl.BlockSpec((Qb, 128), lambda i: (i, 0)),
        pl.BlockSpec((Qb, 8), lambda i: (i, 0)),
        pl.BlockSpec((8, 128), lambda i: (0, 0)),
        pl.BlockSpec((128, 128), lambda i: (0, 0)),
        pl.BlockSpec((1, 128), lambda i: (0, 0)),
        pl.BlockSpec((Qb, K), lambda i: (i, 0)),
        pl.BlockSpec((Qb, 512), lambda i: (i, 0)),
        pl.BlockSpec((Qb, Cs), lambda i: (i, 0)),
        pl.BlockSpec((Cs, 128), lambda i: (0, 0)),
        pl.BlockSpec((1, 128), lambda i: (0, 0)),
    ]
    for wt, bt in zip(mlp_ws[0::2], mlp_ws[1::2]):
        specs.append(pl.BlockSpec(wt.shape, lambda i: (0, 0)))
        specs.append(pl.BlockSpec((1, bt.shape[1]), lambda i: (0, 0)))
    return pl.pallas_call(
        _post_dec_mlp_body,
        grid=(Q // Qb,),
        in_specs=specs,
        out_specs=pl.BlockSpec((Qb, 3), lambda i: (i, 0)),
        out_shape=jax.ShapeDtypeStruct((Q, 3), F32),
    )(m, qp8, Wr8, W2, b2, d16, nf3, skip_f, linW, linb, *mlp_ws)


def _post_dec(m, qp8, Wr8, W2, b2, d16, nf3, skip_f, linW, linb, Qb=512):
    Q = m.shape[0]
    Qb = min(Qb, Q)
    Cs = skip_f.shape[1]
    return pl.pallas_call(
        _post_dec_body,
        grid=(Q // Qb,),
        in_specs=[
            pl.BlockSpec((Qb, 128), lambda i: (i, 0)),
            pl.BlockSpec((Qb, 8), lambda i: (i, 0)),
            pl.BlockSpec((8, 128), lambda i: (0, 0)),
            pl.BlockSpec((128, 128), lambda i: (0, 0)),
            pl.BlockSpec((1, 128), lambda i: (0, 0)),
            pl.BlockSpec((Qb, K), lambda i: (i, 0)),
            pl.BlockSpec((Qb, 512), lambda i: (i, 0)),
            pl.BlockSpec((Qb, Cs), lambda i: (i, 0)),
            pl.BlockSpec((Cs, 128), lambda i: (0, 0)),
            pl.BlockSpec((1, 128), lambda i: (0, 0)),
        ],
        out_specs=pl.BlockSpec((Qb, 128), lambda i: (i, 0)),
        out_shape=jax.ShapeDtypeStruct((Q, 128), F32),
    )(m, qp8, Wr8, W2, b2, d16, nf3, skip_f, linW, linb)


def _post_dec_direct(m, qp8, Wr8, W2, b2, d16, nf3, skip, Qb=512):
    Q = m.shape[0]
    Qb = min(Qb, Q)
    return pl.pallas_call(
        _post_dec_direct_body,
        grid=(Q // Qb,),
        in_specs=[
            pl.BlockSpec((Qb, 128), lambda i: (i, 0)),
            pl.BlockSpec((Qb, 8), lambda i: (i, 0)),
            pl.BlockSpec((8, 128), lambda i: (0, 0)),
            pl.BlockSpec((128, 128), lambda i: (0, 0)),
            pl.BlockSpec((1, 128), lambda i: (0, 0)),
            pl.BlockSpec((Qb, K), lambda i: (i, 0)),
            pl.BlockSpec((Qb, 512), lambda i: (i, 0)),
            pl.BlockSpec((Qb, 128), lambda i: (i, 0)),
        ],
        out_specs=pl.BlockSpec((Qb, 128), lambda i: (i, 0)),
        out_shape=jax.ShapeDtypeStruct((Q, 128), F32),
    )(m, qp8, Wr8, W2, b2, d16, nf3, skip)


def _mlp_body(x_ref, w0, b0, w1, b1, w2, b2, w3, b3, o_ref):
    h = jnp.maximum(jnp.dot(x_ref[...], w0[...], preferred_element_type=F32)
                    + b0[...], 0.0)
    h = jnp.maximum(jnp.dot(h, w1[...], preferred_element_type=F32)
                    + b1[...], 0.0)
    h = jnp.maximum(jnp.dot(h, w2[...], preferred_element_type=F32)
                    + b2[...], 0.0)
    o_ref[...] = jnp.dot(h, w3[...], preferred_element_type=F32) + b3[...]


def _mlp(x, ws, Qb=1024):
    Q = x.shape[0]
    w0, b0, w1, b1, w2, b2, w3, b3 = ws
    specs = [pl.BlockSpec((Qb, 128), lambda i: (i, 0))]
    for wt, bt in ((w0, b0), (w1, b1), (w2, b2), (w3, b3)):
        specs.append(pl.BlockSpec(wt.shape, lambda i: (0, 0)))
        specs.append(pl.BlockSpec((1, bt.shape[1]), lambda i: (0, 0)))
    return pl.pallas_call(
        _mlp_body,
        grid=(Q // Qb,),
        in_specs=specs,
        out_specs=pl.BlockSpec((Qb, 3), lambda i: (i, 0)),
        out_shape=jax.ShapeDtypeStruct((Q, 3), F32),
    )(x, w0, b0, w1, b1, w2, b2, w3, b3)


def _sc_chunk_max(rows_v, mbuf, cq, D):
    def qbody(q, _):
        for c in range(D // 16):
            acc = rows_v[q * K, pl.ds(c * 16, 16)]
            for k in range(1, K):
                acc = jnp.maximum(acc, rows_v[q * K + k, pl.ds(c * 16, 16)])
            mbuf[q, pl.ds(c * 16, 16)] = acc
        return 0
    lax.fori_loop(0, cq, qbody, 0, unroll=False)


def _sc_gather_max_enc(table, idx_flat, D):
    Qt = idx_flat.shape[0]
    Q = Qt // K
    nq = Q // NW
    cq = min(nq, 32)
    nchunks = nq // cq
    mesh = plsc.VectorSubcoreMesh(core_axis_name="c", subcore_axis_name="s")

    @functools.partial(
        pl.kernel, mesh=mesh,
        out_type=jax.ShapeDtypeStruct((Q, D), F32),
        scratch_types=[
            pltpu.VMEM((cq * K,), I32),
            pltpu.VMEM((cq * K, D), F32),
            pltpu.VMEM((cq, D), F32),
            pltpu.SemaphoreType.DMA,
        ],
    )
    def k(table_hbm, idx_hbm, m_hbm, idx_v, rows_v, mbuf, sem):
        wid = lax.axis_index("s") * 2 + lax.axis_index("c")

        def chunk(ch, _):
            base = wid * nq + ch * cq
            pltpu.sync_copy(idx_hbm.at[pl.ds(base * K, cq * K)], idx_v)
            pltpu.async_copy(table_hbm.at[idx_v], rows_v, sem).wait()
            _sc_chunk_max(rows_v, mbuf, cq, D)
            pltpu.sync_copy(mbuf, m_hbm.at[pl.ds(base, cq)])
            return 0

        lax.fori_loop(0, nchunks, chunk, 0, unroll=False)

    return k(table, idx_flat)


def _sc_gather_max_dec(table, table3, idx_flat, idx4_flat, D):
    Qt = idx_flat.shape[0]
    Q = Qt // K
    nq = Q // NW
    cq = min(nq, 16)
    nchunks = nq // cq
    mesh = plsc.VectorSubcoreMesh(core_axis_name="c", subcore_axis_name="s")

    @functools.partial(
        pl.kernel, mesh=mesh,
        out_type=(jax.ShapeDtypeStruct((Q, D), F32),
                  jax.ShapeDtypeStruct((Q * 4, D), F32)),
        scratch_types=[
            pltpu.VMEM((cq * K,), I32),
            pltpu.VMEM((cq * K,), I32),
            pltpu.VMEM((cq * K, D), F32),
            pltpu.VMEM((cq * K, D), F32),
            pltpu.VMEM((cq, D), F32),
            pltpu.VMEM((cq * 4,), I32),
            pltpu.VMEM((cq * 4,), I32),
            pltpu.VMEM((cq * 4, D), F32),
            pltpu.VMEM((cq * 4, D), F32),
            pltpu.SemaphoreType.DMA,
            pltpu.SemaphoreType.DMA,
            pltpu.SemaphoreType.DMA,
            pltpu.SemaphoreType.DMA,
        ],
    )
    def k(table_hbm, table3_hbm, idx_hbm, idx4_hbm, m_hbm, nf4_hbm,
          idx_a, idx_b, rows_a, rows_b, mbuf, idx4_a, idx4_b,
          rows4_a, rows4_b, sem_a, sem_b, sem4_a, sem4_b):
        wid = lax.axis_index("s") * 2 + lax.axis_index("c")
        idxs = (idx_a, idx_b)
        idx4s = (idx4_a, idx4_b)
        rows = (rows_a, rows_b)
        rows4 = (rows4_a, rows4_b)
        sems = (sem_a, sem_b)
        sems4 = (sem4_a, sem4_b)

        def stage(ch, slot):
            base = wid * nq + ch * cq
            pltpu.sync_copy(idx_hbm.at[pl.ds(base * K, cq * K)], idxs[slot])
            pltpu.sync_copy(idx4_hbm.at[pl.ds(base * 4, cq * 4)],
                            idx4s[slot])
            pltpu.async_copy(table_hbm.at[idxs[slot]], rows[slot],
                             sems[slot])
            pltpu.async_copy(table3_hbm.at[idx4s[slot]], rows4[slot],
                             sems4[slot])

        def drain_compute(ch, slot):
            base = wid * nq + ch * cq
            pltpu.make_async_copy(table_hbm.at[idxs[slot]], rows[slot],
                                  sems[slot]).wait()
            pltpu.make_async_copy(table3_hbm.at[idx4s[slot]],
                                  rows4[slot], sems4[slot]).wait()
            _sc_chunk_max(rows[slot], mbuf, cq, D)
            pltpu.sync_copy(mbuf, m_hbm.at[pl.ds(base, cq)])
            pltpu.sync_copy(rows4[slot], nf4_hbm.at[pl.ds(base * 4, cq * 4)])

        if nchunks == 1:
            stage(0, 0)
            drain_compute(0, 0)
        else:
            stage(0, 0)

            def pair(i, _):
                ch0 = i * 2
                stage(ch0 + 1, 1)
                drain_compute(ch0, 0)
                nxt = jnp.minimum(ch0 + 2, nchunks - 1)
                stage(nxt, 0)
                drain_compute(ch0 + 1, 1)
                return 0

            lax.fori_loop(0, nchunks // 2, pair, 0, unroll=False)
            pltpu.make_async_copy(table_hbm.at[idxs[0]], rows[0],
                                  sems[0]).wait()
            pltpu.make_async_copy(table3_hbm.at[idx4s[0]], rows4[0],
                                  sems4[0]).wait()

    return k(table, table3, idx_flat, idx4_flat)


def _pad8(x):
    return jnp.pad(x, ((0, 0), (0, 8 - x.shape[1])))


def _split_W1(W1, C):
    Wf = W1[:C]
    if C == 6:
        Wf = jnp.pad(Wf, ((0, 2), (0, 0)))
    Wr8 = jnp.pad(W1[C:], ((0, 5), (0, 0)))
    return Wf, Wr8


def kernel(points, features, enc0_W1, enc0_b1, enc0_W2, enc0_b2, enc1_W1, enc1_b1, enc1_W2, enc1_b2, enc2_W1, enc2_b1, enc2_W2, enc2_b2, enc3_W1, enc3_b1, enc3_W2, enc3_b2, up0_W1, up0_b1, up0_W2, up0_b2, up1_W1, up1_b1, up1_W2, up1_b2, up2_W1, up2_b1, up2_W2, up2_b2, up3_W1, up3_b1, up3_W2, up3_b2, lin0_W, lin0_b, lin1_W, lin1_b, lin2_W, lin2_b, mlp0_W, mlp0_b, mlp1_W, mlp1_b, mlp2_W, mlp2_b, mlp3_W, mlp3_b):
    r1 = lambda b: b.reshape(1, -1)
    pts8 = _pad8(points)
    feat8 = _pad8(features)
    q0 = pts8[::4]
    q1 = q0[::4]
    q2 = q1[::4]
    ptsT = pts8.T
    q0T = q0.T
    q1T = q1.T
    q2T = q2.T

    i0, _, _ = _knn(q0, ptsT)
    i1, _, _ = _knn(q1, q0T)
    i2, _, _ = _knn(q2, q1T)
    i3, d3, i3_4 = _knn(q2, q2T)
    iu2, du2, iu2_4 = _knn(q1, q2T)
    iu1, du1, iu1_4 = _knn(q0, q1T)
    iu0, du0, iu0_4 = _knn(pts8, q0T)

    Wf, Wr = _split_W1(enc0_W1, 6)
    c = _prep(feat8, pts8, jnp.pad(enc0_W1[:6], ((0, 2), (0, 0))), Wr,
              r1(enc0_b1), False)
    m = _sc_gather_max_enc(c, i0.reshape(-1), 128)
    f0 = _post_enc(m, q0, Wr, enc0_W2, r1(enc0_b2))

    Wf, Wr = _split_W1(enc1_W1, 64)
    c = _prep(f0, q0, Wf, Wr, r1(enc1_b1), True)
    m = _sc_gather_max_enc(c, i1.reshape(-1), 128)
    f1 = _post_enc(m, q1, Wr, enc1_W2, r1(enc1_b2))

    Wf, Wr = _split_W1(enc2_W1, 96)
    c = _prep(f1, q1, Wf, Wr, r1(enc2_b1), True)
    m = _sc_gather_max_enc(c, i2.reshape(-1), 128)
    f2 = _post_enc(m, q2, Wr, enc2_W2, r1(enc2_b2))

    Wf, Wr = _split_W1(enc3_W1, 128)
    c = _prep(f2, q2, Wf, Wr, r1(enc3_b1), True)
    m = _sc_gather_max_enc(c, i3.reshape(-1), 128)
    f3 = _post_enc(m, q2, Wr, enc3_W2, r1(enc3_b2))

    Wf, Wr = _split_W1(up3_W1, 128)
    c = _prep(f3, q2, Wf, Wr, r1(up3_b1), True)
    m, nf4 = _sc_gather_max_dec(c, f3, i3.reshape(-1), i3_4.reshape(-1), 128)
    fe = _post_dec_direct(m, q2, Wr, up3_W2, r1(up3_b2), d3,
                          nf4.reshape(-1, 512), f2)

    Wf, Wr = _split_W1(up2_W1, 128)
    c = _prep(fe, q2, Wf, Wr, r1(up2_b1), True)
    m, nf4 = _sc_gather_max_dec(c, fe, iu2.reshape(-1), iu2_4.reshape(-1), 128)
    fe = _post_dec(m, q1, Wr, up2_W2, r1(up2_b2), du2,
                   nf4.reshape(-1, 512), f1, lin2_W, r1(lin2_b))

    Wf, Wr = _split_W1(up1_W1, 128)
    c = _prep(fe, q1, Wf, Wr, r1(up1_b1), True)
    m, nf4 = _sc_gather_max_dec(c, fe, iu1.reshape(-1), iu1_4.reshape(-1), 128)
    fe = _post_dec(m, q0, Wr, up1_W2, r1(up1_b2), du1,
                   nf4.reshape(-1, 512), f0, lin1_W, r1(lin1_b))

    Wf, Wr = _split_W1(up0_W1, 128)
    c = _prep(fe, q0, Wf, Wr, r1(up0_b1), True)
    m, nf4 = _sc_gather_max_dec(c, fe, iu0.reshape(-1), iu0_4.reshape(-1), 128)
    return _post_dec_mlp(m, pts8, Wr, up0_W2, r1(up0_b2), du0,
                         nf4.reshape(-1, 512), feat8,
                         jnp.pad(lin0_W, ((0, 2), (0, 0))), r1(lin0_b),
                         (mlp0_W, r1(mlp0_b), mlp1_W, r1(mlp1_b),
                          mlp2_W, r1(mlp2_b), mlp3_W, r1(mlp3_b)))
